# baseline (speedup 1.0000x reference)
import functools

import jax
import jax.numpy as jnp
from jax import lax
from jax.experimental import pallas as pl
from jax.experimental.pallas import tpu as pltpu
from jax.experimental.pallas import tpu_sc as plsc

_B = 16
_L = 4096
_D = 512
_TOTAL = _B * _L // 2
_NW = 32
_RPW = _B * _L // _NW
_CHUNK = 64
_NCHUNK = _B * _L // _CHUNK
_CPW = _NCHUNK // _NW
_CPS = _L // _CHUNK
_LANES = 16


def _sc_body(flat_hbm, starts_hbm, ends_hbm, out_hbm, mask_hbm,
             st_v, en_v, idx0, idx1, buf0, buf1, zbuf, mbuf,
             isem0, isem1, osem0, osem1, zsem):
    cid = lax.axis_index("c")
    sid = lax.axis_index("s")
    w = sid * 2 + cid

    pltpu.sync_copy(starts_hbm, st_v.at[pl.ds(0, _LANES)])
    pltpu.sync_copy(ends_hbm, en_v.at[pl.ds(0, _LANES)])
    iota = lax.iota(jnp.int32, _LANES)
    zerosf = jnp.zeros((_LANES,), jnp.float32)

    bufs = (buf0, buf1)
    idxs = (idx0, idx1)
    isems = (isem0, isem1)
    osems = (osem0, osem1)

    def _zrow(row, _):
        for kk in range(_D // _LANES):
            zbuf[row, pl.ds(kk * _LANES, _LANES)] = zerosf
        return 0
    lax.fori_loop(0, _CHUNK, _zrow, 0)

    mb = w // 2
    mt0 = (w % 2) * _RPW
    mstart = st_v[pl.ds(mb, _LANES)][0]
    mend = en_v[pl.ds(mb, _LANES)][0]
    mnv = jnp.clip(mend - mstart - mt0, 0, _RPW)
    for j in range(_RPW // _LANES):
        m = jnp.where(j * _LANES + iota < mnv, 1.0, 0.0).astype(jnp.float32)
        mbuf[pl.ds(j * _LANES, _LANES)] = m
    pltpu.sync_copy(
        mbuf, mask_hbm.at[pl.ds(pl.multiple_of(w * _RPW, _RPW), _RPW)])

    def _slot(i, bi, used_bi, nz):
        c_glob = i * _NW + w
        bseg = c_glob // _CPS
        trow = (c_glob % _CPS) * _CHUNK
        st = st_v[pl.ds(bseg, _LANES)][0]
        en = en_v[pl.ds(bseg, _LANES)][0]
        nvc = jnp.clip(en - st - trow, 0, _CHUNK)
        s = st + trow
        dst = pl.multiple_of(c_glob * _CHUNK, _CHUNK)

        @pl.when(nvc > 0)
        def _():
            @pl.when(used_bi > 0)
            def _():
                pltpu.make_async_copy(
                    bufs[bi], out_hbm.at[pl.ds(0, _CHUNK)], osems[bi]
                ).wait()

            for kk in range(_CHUNK // _LANES):
                v = jnp.minimum(s + kk * _LANES + iota, _TOTAL - 1)
                idxs[bi][pl.ds(kk * _LANES, _LANES)] = v
            pltpu.make_async_copy(
                flat_hbm.at[idxs[bi]], bufs[bi], isems[bi]).start()
            pltpu.make_async_copy(
                flat_hbm.at[idxs[bi]], bufs[bi], isems[bi]).wait()

            def _ztail(row, _c):
                for kk in range(_D // _LANES):
                    bufs[bi][row, pl.ds(kk * _LANES, _LANES)] = zerosf
                return 0
            lax.fori_loop(nvc, _CHUNK, _ztail, 0)

            pltpu.make_async_copy(
                bufs[bi], out_hbm.at[pl.ds(dst, _CHUNK)], osems[bi]).start()

        @pl.when(nvc == 0)
        def _():
            pltpu.make_async_copy(
                zbuf, out_hbm.at[pl.ds(dst, _CHUNK)], zsem).start()

        used_new = jnp.where(nvc > 0, jnp.int32(1), used_bi)
        nz_new = nz + jnp.where(nvc == 0, jnp.int32(1), jnp.int32(0))
        return used_new, nz_new

    def _pair(g, carry):
        u0, u1, nz = carry
        u0, nz = _slot(g * 2, 0, u0, nz)
        u1, nz = _slot(g * 2 + 1, 1, u1, nz)
        return u0, u1, nz

    u0, u1, nz = lax.fori_loop(
        0, _CPW // 2, _pair,
        (jnp.int32(0), jnp.int32(0), jnp.int32(0)))

    for bi, u in ((0, u0), (1, u1)):
        @pl.when(u > 0)
        def _(bi=bi):
            pltpu.make_async_copy(
                bufs[bi], out_hbm.at[pl.ds(0, _CHUNK)], osems[bi]
            ).wait()

    def _zdrain(h, _):
        pltpu.make_async_copy(
            zbuf, out_hbm.at[pl.ds(0, _CHUNK)], zsem).wait()
        return 0
    lax.fori_loop(0, nz, _zdrain, 0)


@jax.jit
def _padded_gather(flat, starts, ends):
    mesh = plsc.VectorSubcoreMesh(core_axis_name="c", subcore_axis_name="s")
    return pl.kernel(
        _sc_body,
        out_type=(
            jax.ShapeDtypeStruct((_B * _L, _D), jnp.float32),
            jax.ShapeDtypeStruct((_B * _L,), jnp.float32),
        ),
        mesh=mesh,
        scratch_types=[
            pltpu.VMEM((2 * _LANES,), jnp.int32),
            pltpu.VMEM((2 * _LANES,), jnp.int32),
            pltpu.VMEM((_CHUNK,), jnp.int32),
            pltpu.VMEM((_CHUNK,), jnp.int32),
            pltpu.VMEM((_CHUNK, _D), jnp.float32),
            pltpu.VMEM((_CHUNK, _D), jnp.float32),
            pltpu.VMEM((_CHUNK, _D), jnp.float32),
            pltpu.VMEM((_RPW,), jnp.float32),
            pltpu.SemaphoreType.DMA,
            pltpu.SemaphoreType.DMA,
            pltpu.SemaphoreType.DMA,
            pltpu.SemaphoreType.DMA,
            pltpu.SemaphoreType.DMA,
        ],
    )(flat, starts, ends)


def kernel(flat, cu_seqlens):
    starts = cu_seqlens[:-1]
    ends = cu_seqlens[1:]
    return _padded_gather(flat, starts, ends)
